# baseline (speedup 1.0000x reference)
import functools

import jax
import jax.numpy as jnp
from jax import lax
from jax.experimental import pallas as pl
from jax.experimental.pallas import tpu as pltpu
from jax.experimental.pallas import tpu_sc as plsc

L = 16
NC = 2
NS = 16
NW = NC * NS
RCH = 128
QUAD = 4
UNROLL = 4


@functools.lru_cache(maxsize=None)
def _make_fused(M, D, B):
    RPW8 = (-(-M // NW) + 7) // 8 * 8
    TAIL = M - (NW - 1) * RPW8
    SUB = RPW8 - TAIL
    NV = B // L
    POSV = (RPW8 + L - 1) // L
    GROUP = QUAD * RCH
    NCHMAX = (RPW8 + GROUP - 1) // GROUP * QUAD
    CAP = NCHMAX * RCH + L
    assert TAIL > 0 and TAIL % 8 == 0 and (SUB == 0 or SUB % 8 == 0)

    mesh = plsc.VectorSubcoreMesh(
        core_axis_name="c", subcore_axis_name="s", num_cores=NC, num_subcores=NS
    )

    @functools.partial(
        pl.kernel,
        out_type=jax.ShapeDtypeStruct((M, D), jnp.float32),
        mesh=mesh,
        compiler_params=pltpu.CompilerParams(needs_layout_passes=False),
        scratch_types=[
            pltpu.VMEM((B,), jnp.int32),
            pltpu.VMEM((POSV * L,), jnp.int32),
            pltpu.VMEM((CAP,), jnp.int32),
            pltpu.VMEM((CAP,), jnp.int32),
            pltpu.VMEM((NCHMAX, RCH), jnp.int32),
            pltpu.VMEM((NCHMAX, RCH), jnp.int32),
        ]
        + [pltpu.VMEM((RCH, D), jnp.float32) for _ in range(QUAD)]
        + [
            pltpu.SemaphoreType.DMA,
            pltpu.SemaphoreType.DMA,
            pltpu.SemaphoreType.DMA,
        ],
    )
    def fused_kernel(mem_hbm, idx_hbm, val_hbm, out_hbm, idxf, pos, cidx1,
                     cpos1, cidx2, cpos2, *rest):
        rows = rest[:QUAD]
        csem, gsem, ssem = rest[QUAD:]
        c = lax.axis_index("c")
        s = lax.axis_index("s")
        wid = s * NC + c
        base = wid * RPW8
        rpw_eff = jnp.where(wid == NW - 1, TAIL, RPW8)
        lane = lax.iota(jnp.int32, L)

        pltpu.sync_copy(idx_hbm, idxf)
        copy_a = pltpu.async_copy(
            mem_hbm.at[pl.ds(base, TAIL)], out_hbm.at[pl.ds(base, TAIL)], csem
        )
        off_b = jnp.minimum(base + TAIL, M - SUB)
        copy_b = pltpu.async_copy(
            mem_hbm.at[pl.ds(off_b, SUB)], out_hbm.at[pl.ds(off_b, SUB)], csem
        )

        neg1 = jnp.full((L,), -1, jnp.int32)
        for t in range(POSV):
            pos[pl.ds(t * L, L)] = neg1

        def p1_body(k, _):
            for u in range(UNROLL):
                off = (k * UNROLL + u) * L
                v = idxf[pl.ds(off, L)]
                vl = v - base
                m = (vl >= 0) & (vl < rpw_eff)
                vc = jnp.clip(vl, 0, RPW8 - 1)
                plsc.store_scatter(pos, [vc], off + lane, mask=m)
            return 0

        lax.fori_loop(0, NV // UNROLL, p1_body, 0)

        def fix_body(k, _):
            for u in range(UNROLL):
                off = (k * UNROLL + u) * L
                v = idxf[pl.ds(off, L)]
                vl = v - base
                m = (vl >= 0) & (vl < rpw_eff)
                vc = jnp.clip(vl, 0, RPW8 - 1)
                b = off + lane
                p = plsc.load_gather(pos, [vc])
                plsc.store_scatter(pos, [vc], b, mask=m & (p < b))
            return 0

        lax.fori_loop(0, NV // UNROLL, fix_body, 0)
        lax.fori_loop(0, NV // UNROLL, fix_body, 0)
        lax.fori_loop(0, NV // UNROLL, fix_body, 0)

        def ext_body(t, cursor):
            p = pos[pl.ds(t * L, L)]
            m = p >= jnp.int32(0)
            r = base + t * L + lane
            plsc.store_compressed(cidx1.at[pl.ds(cursor, L)], r, mask=m)
            plsc.store_compressed(cpos1.at[pl.ds(cursor, L)], p, mask=m)
            return cursor + jnp.sum(m.astype(jnp.int32))

        n_win = lax.fori_loop(0, POSV, ext_body, jnp.int32(0))

        target = (n_win + GROUP - 1) // GROUP * GROUP
        v0 = cidx1[pl.ds(0, L)]
        p0 = cpos1[pl.ds(0, L)]
        is0 = lane == 0
        padi = jnp.sum(jnp.where(is0, v0, 0))
        padp = jnp.sum(jnp.where(is0, p0, 0))
        padiv = jnp.full((L,), padi, jnp.int32)
        padpv = jnp.full((L,), padp, jnp.int32)

        def pad_body(t, _):
            cidx1[pl.ds(n_win + t * L, L)] = padiv
            cpos1[pl.ds(n_win + t * L, L)] = padpv
            return 0

        lax.fori_loop(0, (target - n_win + L - 1) // L, pad_body, 0)

        for j in range(NCHMAX):
            for t in range(RCH // L):
                cidx2[j, pl.ds(t * L, L)] = cidx1[pl.ds(j * RCH + t * L, L)]
                cpos2[j, pl.ds(t * L, L)] = cpos1[pl.ds(j * RCH + t * L, L)]

        copy_a.wait()
        copy_b.wait()

        def quad_body(q, _):
            gathers = [
                pltpu.async_copy(
                    val_hbm.at[cpos2.at[q * QUAD + u]], rows[u], gsem
                )
                for u in range(QUAD)
            ]
            scatters = []
            for u in range(QUAD):
                gathers[u].wait()
                scatters.append(
                    pltpu.async_copy(
                        rows[u], out_hbm.at[cidx2.at[q * QUAD + u]], ssem
                    )
                )
            for sc_ in scatters:
                sc_.wait()
            return 0

        lax.fori_loop(0, target // GROUP, quad_body, 0)

    return fused_kernel


def kernel(mem, idx, val):
    M, D = mem.shape
    B = idx.shape[0]
    return _make_fused(M, D, B)(mem, idx, val)


# speedup vs baseline: 14.7463x; 14.7463x over previous
import functools

import jax
import jax.numpy as jnp
from jax import lax
from jax.experimental import pallas as pl
from jax.experimental.pallas import tpu as pltpu
from jax.experimental.pallas import tpu_sc as plsc

L = 16
NC = 2
NS = 16
NW = NC * NS
RCH = 128
QUAD = 2
UNROLL = 4
CROWS = 120
NBUF = 4


@functools.lru_cache(maxsize=None)
def _make_fused(M, D, B):
    RPW8 = (-(-M // NW) + 7) // 8 * 8
    TAIL = M - (NW - 1) * RPW8
    NV = B // L
    POSV = (RPW8 + L - 1) // L
    GROUP = QUAD * RCH
    NCHMAX = (RPW8 + GROUP - 1) // GROUP * QUAD
    CAP = NCHMAX * RCH + L
    assert TAIL > 0 and TAIL % 8 == 0

    copy_chunks = []
    off = 0
    while off < RPW8:
        sz = min(CROWS, RPW8 - off)
        assert sz % 8 == 0
        copy_chunks.append((off, sz))
        off += sz
    NCH = len(copy_chunks)

    mesh = plsc.VectorSubcoreMesh(
        core_axis_name="c", subcore_axis_name="s", num_cores=NC, num_subcores=NS
    )

    @functools.partial(
        pl.kernel,
        out_type=jax.ShapeDtypeStruct((M, D), jnp.float32),
        mesh=mesh,
        compiler_params=pltpu.CompilerParams(needs_layout_passes=False),
        scratch_types=[
            pltpu.VMEM((B,), jnp.int32),
            pltpu.VMEM((POSV * L,), jnp.int32),
            pltpu.VMEM((CAP,), jnp.int32),
            pltpu.VMEM((CAP,), jnp.int32),
            pltpu.VMEM((NCHMAX, RCH), jnp.int32),
            pltpu.VMEM((NCHMAX, RCH), jnp.int32),
            pltpu.VMEM_SHARED((NS, NBUF, CROWS, D), jnp.float32),
        ]
        + [pltpu.VMEM((RCH, D), jnp.float32) for _ in range(QUAD)]
        + [pltpu.SemaphoreType.DMA for _ in range(NBUF)]
        + [pltpu.SemaphoreType.DMA for _ in range(NBUF)]
        + [
            pltpu.SemaphoreType.DMA,
            pltpu.SemaphoreType.DMA,
        ],
    )
    def fused_kernel(mem_hbm, idx_hbm, val_hbm, out_hbm, idxf, pos, cidx1,
                     cpos1, cidx2, cpos2, ring, *rest):
        rows = rest[:QUAD]
        rsem = rest[QUAD:QUAD + NBUF]
        wsem = rest[QUAD + NBUF:QUAD + 2 * NBUF]
        gsem, ssem = rest[QUAD + 2 * NBUF:]
        c = lax.axis_index("c")
        s = lax.axis_index("s")
        wid = s * NC + c
        base = wid * RPW8
        rpw_eff = jnp.where(wid == NW - 1, TAIL, RPW8)
        lane = lax.iota(jnp.int32, L)

        pltpu.sync_copy(idx_hbm, idxf)

        def chunk_off(i):
            rel, sz = copy_chunks[i]
            return jnp.minimum(base + rel, M - sz)

        rdesc = {}
        wdesc = {}

        def fire_read(i):
            _, sz = copy_chunks[i]
            o = chunk_off(i)
            rdesc[i] = pltpu.async_copy(
                mem_hbm.at[pl.ds(o, sz)],
                ring.at[s, i % NBUF, pl.ds(0, sz)],
                rsem[i % NBUF],
            )

        def fire_write(i):
            _, sz = copy_chunks[i]
            o = chunk_off(i)
            wdesc[i] = pltpu.async_copy(
                ring.at[s, i % NBUF, pl.ds(0, sz)],
                out_hbm.at[pl.ds(o, sz)],
                wsem[i % NBUF],
            )

        state = {"next_read": 0, "write_q": [], "refill_q": [], "pt": 0}

        def prime_ring():
            for _ in range(NBUF):
                if state["next_read"] < NCH:
                    i = state["next_read"]
                    fire_read(i)
                    state["write_q"].append((i, state["pt"]))
                    state["next_read"] += 1

        def service_point(k=2):
            state["pt"] += 1
            for _ in range(k):
                if state["refill_q"] and state["refill_q"][0][1] < state["pt"]:
                    i, _p = state["refill_q"].pop(0)
                    wdesc[i].wait()
                    if state["next_read"] < NCH:
                        j = state["next_read"]
                        fire_read(j)
                        state["write_q"].append((j, state["pt"]))
                        state["next_read"] += 1
            for _ in range(k):
                if state["write_q"] and state["write_q"][0][1] < state["pt"]:
                    i, _p = state["write_q"].pop(0)
                    rdesc[i].wait()
                    fire_write(i)
                    state["refill_q"].append((i, state["pt"]))

        def drain_ring():
            while state["write_q"]:
                i, _p = state["write_q"].pop(0)
                rdesc[i].wait()
                fire_write(i)
                state["refill_q"].append((i, 0))
            while state["refill_q"]:
                wdesc[state["refill_q"].pop(0)[0]].wait()

        prime_ring()

        neg1 = jnp.full((L,), -1, jnp.int32)
        for t in range(POSV):
            pos[pl.ds(t * L, L)] = neg1

        def p1_body(k, _):
            for u in range(UNROLL):
                off = (k * UNROLL + u) * L
                v = idxf[pl.ds(off, L)]
                vl = v - base
                m = (vl >= 0) & (vl < rpw_eff)
                vc = jnp.clip(vl, 0, RPW8 - 1)
                plsc.store_scatter(pos, [vc], off + lane, mask=m)
            return 0

        def fix_body(k, _):
            for u in range(UNROLL):
                off = (k * UNROLL + u) * L
                v = idxf[pl.ds(off, L)]
                vl = v - base
                m = (vl >= 0) & (vl < rpw_eff)
                vc = jnp.clip(vl, 0, RPW8 - 1)
                b = off + lane
                p = plsc.load_gather(pos, [vc])
                plsc.store_scatter(pos, [vc], b, mask=m & (p < b))
            return 0

        NQ = NV // UNROLL // 4
        for q in range(4):
            lax.fori_loop(q * NQ, (q + 1) * NQ, p1_body, 0)
            service_point()
        for _ in range(3):
            for q in range(4):
                lax.fori_loop(q * NQ, (q + 1) * NQ, fix_body, 0)
                service_point()

        def ext_body(t, cursor):
            p = pos[pl.ds(t * L, L)]
            m = p >= jnp.int32(0)
            r = base + t * L + lane
            plsc.store_compressed(cidx1.at[pl.ds(cursor, L)], r, mask=m)
            plsc.store_compressed(cpos1.at[pl.ds(cursor, L)], p, mask=m)
            return cursor + jnp.sum(m.astype(jnp.int32))

        n_win = lax.fori_loop(0, POSV, ext_body, jnp.int32(0))
        service_point()

        target = (n_win + GROUP - 1) // GROUP * GROUP
        v0 = cidx1[pl.ds(0, L)]
        p0 = cpos1[pl.ds(0, L)]
        is0 = lane == 0
        padi = jnp.sum(jnp.where(is0, v0, 0))
        padp = jnp.sum(jnp.where(is0, p0, 0))
        padiv = jnp.full((L,), padi, jnp.int32)
        padpv = jnp.full((L,), padp, jnp.int32)

        def pad_body(t, _):
            cidx1[pl.ds(n_win + t * L, L)] = padiv
            cpos1[pl.ds(n_win + t * L, L)] = padpv
            return 0

        lax.fori_loop(0, (target - n_win + L - 1) // L, pad_body, 0)
        service_point()

        for j in range(NCHMAX):
            for t in range(RCH // L):
                cidx2[j, pl.ds(t * L, L)] = cidx1[pl.ds(j * RCH + t * L, L)]
                cpos2[j, pl.ds(t * L, L)] = cpos1[pl.ds(j * RCH + t * L, L)]

        drain_ring()

        def quad_body(q, _):
            gathers = [
                pltpu.async_copy(
                    val_hbm.at[cpos2.at[q * QUAD + u]], rows[u], gsem
                )
                for u in range(QUAD)
            ]
            scatters = []
            for u in range(QUAD):
                gathers[u].wait()
                scatters.append(
                    pltpu.async_copy(
                        rows[u], out_hbm.at[cidx2.at[q * QUAD + u]], ssem
                    )
                )
            for sc_ in scatters:
                sc_.wait()
            return 0

        lax.fori_loop(0, target // GROUP, quad_body, 0)

    return fused_kernel


def kernel(mem, idx, val):
    M, D = mem.shape
    B = idx.shape[0]
    return _make_fused(M, D, B)(mem, idx, val)


# speedup vs baseline: 14.9611x; 1.0146x over previous
import functools

import jax
import jax.numpy as jnp
from jax import lax
from jax.experimental import pallas as pl
from jax.experimental.pallas import tpu as pltpu
from jax.experimental.pallas import tpu_sc as plsc

L = 16
NC = 2
NS = 16
NW = NC * NS
RCH = 128
QUAD = 2
UNROLL = 4
CROWS = 240
NBUF = 2


@functools.lru_cache(maxsize=None)
def _make_fused(M, D, B):
    RPW8 = (-(-M // NW) + 7) // 8 * 8
    TAIL = M - (NW - 1) * RPW8
    NV = B // L
    POSV = (RPW8 + L - 1) // L
    GROUP = QUAD * RCH
    NCHMAX = (RPW8 + GROUP - 1) // GROUP * QUAD
    CAP = NCHMAX * RCH + L
    assert TAIL > 0 and TAIL % 8 == 0

    copy_chunks = []
    off = 0
    while off < RPW8:
        sz = min(CROWS, RPW8 - off)
        assert sz % 8 == 0
        copy_chunks.append((off, sz))
        off += sz
    NCH = len(copy_chunks)

    mesh = plsc.VectorSubcoreMesh(
        core_axis_name="c", subcore_axis_name="s", num_cores=NC, num_subcores=NS
    )

    @functools.partial(
        pl.kernel,
        out_type=jax.ShapeDtypeStruct((M, D), jnp.float32),
        mesh=mesh,
        compiler_params=pltpu.CompilerParams(needs_layout_passes=False),
        scratch_types=[
            pltpu.VMEM((B,), jnp.int32),
            pltpu.VMEM((POSV * L,), jnp.int32),
            pltpu.VMEM((CAP,), jnp.int32),
            pltpu.VMEM((CAP,), jnp.int32),
            pltpu.VMEM((NCHMAX, RCH), jnp.int32),
            pltpu.VMEM((NCHMAX, RCH), jnp.int32),
            pltpu.VMEM_SHARED((NS, NBUF, CROWS, D), jnp.float32),
        ]
        + [pltpu.VMEM((RCH, D), jnp.float32) for _ in range(QUAD)]
        + [pltpu.SemaphoreType.DMA for _ in range(NBUF)]
        + [pltpu.SemaphoreType.DMA for _ in range(NBUF)]
        + [
            pltpu.SemaphoreType.DMA,
            pltpu.SemaphoreType.DMA,
        ],
    )
    def fused_kernel(mem_hbm, idx_hbm, val_hbm, out_hbm, idxf, pos, cidx1,
                     cpos1, cidx2, cpos2, ring, *rest):
        rows = rest[:QUAD]
        rsem = rest[QUAD:QUAD + NBUF]
        wsem = rest[QUAD + NBUF:QUAD + 2 * NBUF]
        gsem, ssem = rest[QUAD + 2 * NBUF:]
        c = lax.axis_index("c")
        s = lax.axis_index("s")
        wid = s * NC + c
        base = wid * RPW8
        rpw_eff = jnp.where(wid == NW - 1, TAIL, RPW8)
        lane = lax.iota(jnp.int32, L)

        pltpu.sync_copy(idx_hbm, idxf)

        def chunk_off(i):
            rel, sz = copy_chunks[i]
            return jnp.minimum(base + rel, M - sz)

        rdesc = {}
        wdesc = {}

        def fire_read(i):
            _, sz = copy_chunks[i]
            o = chunk_off(i)
            rdesc[i] = pltpu.async_copy(
                mem_hbm.at[pl.ds(o, sz)],
                ring.at[s, i % NBUF, pl.ds(0, sz)],
                rsem[i % NBUF],
            )

        def fire_write(i):
            _, sz = copy_chunks[i]
            o = chunk_off(i)
            wdesc[i] = pltpu.async_copy(
                ring.at[s, i % NBUF, pl.ds(0, sz)],
                out_hbm.at[pl.ds(o, sz)],
                wsem[i % NBUF],
            )

        state = {"next_read": 0, "write_q": [], "refill_q": [], "pt": 0}

        def prime_ring():
            for _ in range(NBUF):
                if state["next_read"] < NCH:
                    i = state["next_read"]
                    fire_read(i)
                    state["write_q"].append((i, state["pt"]))
                    state["next_read"] += 1

        def service_point(k=2):
            state["pt"] += 1
            for _ in range(k):
                if state["refill_q"] and state["refill_q"][0][1] < state["pt"]:
                    i, _p = state["refill_q"].pop(0)
                    wdesc[i].wait()
                    if state["next_read"] < NCH:
                        j = state["next_read"]
                        fire_read(j)
                        state["write_q"].append((j, state["pt"]))
                        state["next_read"] += 1
            for _ in range(k):
                if state["write_q"] and state["write_q"][0][1] < state["pt"]:
                    i, _p = state["write_q"].pop(0)
                    rdesc[i].wait()
                    fire_write(i)
                    state["refill_q"].append((i, state["pt"]))

        def drain_ring():
            while state["write_q"]:
                i, _p = state["write_q"].pop(0)
                rdesc[i].wait()
                fire_write(i)
                state["refill_q"].append((i, 0))
            while state["refill_q"]:
                wdesc[state["refill_q"].pop(0)[0]].wait()

        prime_ring()

        neg1 = jnp.full((L,), -1, jnp.int32)
        for t in range(POSV):
            pos[pl.ds(t * L, L)] = neg1

        def p1_body(k, _):
            for u in range(UNROLL):
                off = (k * UNROLL + u) * L
                v = idxf[pl.ds(off, L)]
                vl = v - base
                m = (vl >= 0) & (vl < rpw_eff)
                vc = jnp.clip(vl, 0, RPW8 - 1)
                plsc.store_scatter(pos, [vc], off + lane, mask=m)
            return 0

        def fix_body(k, _):
            for u in range(UNROLL):
                off = (k * UNROLL + u) * L
                v = idxf[pl.ds(off, L)]
                vl = v - base
                m = (vl >= 0) & (vl < rpw_eff)
                vc = jnp.clip(vl, 0, RPW8 - 1)
                b = off + lane
                p = plsc.load_gather(pos, [vc])
                plsc.store_scatter(pos, [vc], b, mask=m & (p < b))
            return 0

        NQ = NV // UNROLL // 4
        for q in range(4):
            lax.fori_loop(q * NQ, (q + 1) * NQ, p1_body, 0)
            service_point()
        for _ in range(3):
            for q in range(4):
                lax.fori_loop(q * NQ, (q + 1) * NQ, fix_body, 0)
                service_point()

        def ext_body(t, cursor):
            p = pos[pl.ds(t * L, L)]
            m = p >= jnp.int32(0)
            r = base + t * L + lane
            plsc.store_compressed(cidx1.at[pl.ds(cursor, L)], r, mask=m)
            plsc.store_compressed(cpos1.at[pl.ds(cursor, L)], p, mask=m)
            return cursor + jnp.sum(m.astype(jnp.int32))

        n_win = lax.fori_loop(0, POSV, ext_body, jnp.int32(0))
        service_point()

        target = (n_win + GROUP - 1) // GROUP * GROUP
        v0 = cidx1[pl.ds(0, L)]
        p0 = cpos1[pl.ds(0, L)]
        is0 = lane == 0
        padi = jnp.sum(jnp.where(is0, v0, 0))
        padp = jnp.sum(jnp.where(is0, p0, 0))
        padiv = jnp.full((L,), padi, jnp.int32)
        padpv = jnp.full((L,), padp, jnp.int32)

        def pad_body(t, _):
            cidx1[pl.ds(n_win + t * L, L)] = padiv
            cpos1[pl.ds(n_win + t * L, L)] = padpv
            return 0

        lax.fori_loop(0, (target - n_win + L - 1) // L, pad_body, 0)
        service_point()

        for j in range(NCHMAX):
            for t in range(RCH // L):
                cidx2[j, pl.ds(t * L, L)] = cidx1[pl.ds(j * RCH + t * L, L)]
                cpos2[j, pl.ds(t * L, L)] = cpos1[pl.ds(j * RCH + t * L, L)]

        drain_ring()

        def quad_body(q, _):
            gathers = [
                pltpu.async_copy(
                    val_hbm.at[cpos2.at[q * QUAD + u]], rows[u], gsem
                )
                for u in range(QUAD)
            ]
            scatters = []
            for u in range(QUAD):
                gathers[u].wait()
                scatters.append(
                    pltpu.async_copy(
                        rows[u], out_hbm.at[cidx2.at[q * QUAD + u]], ssem
                    )
                )
            for sc_ in scatters:
                sc_.wait()
            return 0

        lax.fori_loop(0, target // GROUP, quad_body, 0)

    return fused_kernel


def kernel(mem, idx, val):
    M, D = mem.shape
    B = idx.shape[0]
    return _make_fused(M, D, B)(mem, idx, val)


# speedup vs baseline: 15.1480x; 1.0125x over previous
import functools

import jax
import jax.numpy as jnp
from jax import lax
from jax.experimental import pallas as pl
from jax.experimental.pallas import tpu as pltpu
from jax.experimental.pallas import tpu_sc as plsc

L = 16
NC = 2
NS = 16
NW = NC * NS
RCH = 128
UNROLL = 4


@functools.lru_cache(maxsize=None)
def _make_winners(M, B):
    BPW = B // NW
    NV = B // L
    NCHUNK = BPW // RCH
    CAP = BPW + L

    mesh = plsc.VectorSubcoreMesh(
        core_axis_name="c", subcore_axis_name="s", num_cores=NC, num_subcores=NS
    )

    @functools.partial(
        pl.kernel,
        out_type=(
            jax.ShapeDtypeStruct((NW, NCHUNK, RCH), jnp.int32),
            jax.ShapeDtypeStruct((NW, NCHUNK, RCH), jnp.int32),
        ),
        mesh=mesh,
        compiler_params=pltpu.CompilerParams(needs_layout_passes=False),
        scratch_types=[
            pltpu.VMEM((B,), jnp.int32),
            pltpu.VMEM((M,), jnp.int32),
            pltpu.VMEM((CAP,), jnp.int32),
            pltpu.VMEM((CAP,), jnp.int32),
            pltpu.VMEM((NCHUNK, RCH), jnp.int32),
            pltpu.VMEM((NCHUNK, RCH), jnp.int32),
        ],
    )
    def winners_kernel(idx_hbm, cidx_hbm, cpos_hbm, idxf, pos, cidx1, cpos1,
                       cidx2, cpos2):
        c = lax.axis_index("c")
        s = lax.axis_index("s")
        wid = s * NC + c
        lane = lax.iota(jnp.int32, L)

        pltpu.sync_copy(idx_hbm, idxf)

        def p1_body(k, _):
            for u in range(UNROLL):
                off = (k * UNROLL + u) * L
                v = idxf[pl.ds(off, L)]
                plsc.store_scatter(pos, [v], off + lane)
            return 0

        lax.fori_loop(0, NV // UNROLL, p1_body, 0)

        def fix_body(k, _):
            for u in range(UNROLL):
                off = (k * UNROLL + u) * L
                v = idxf[pl.ds(off, L)]
                b = off + lane
                p = plsc.load_gather(pos, [v])
                plsc.store_scatter(pos, [v], b, mask=p < b)
            return 0

        lax.fori_loop(0, NV // UNROLL, fix_body, 0)
        lax.fori_loop(0, NV // UNROLL, fix_body, 0)

        ebase = wid * BPW

        def ext_body(k, cursor):
            v = idxf[pl.ds(ebase + k * L, L)]
            b = ebase + k * L + lane
            p = plsc.load_gather(pos, [v])
            m = p == b
            plsc.store_compressed(cidx1.at[pl.ds(cursor, L)], v, mask=m)
            plsc.store_compressed(cpos1.at[pl.ds(cursor, L)], b, mask=m)
            return cursor + jnp.sum(m.astype(jnp.int32))

        n_win = lax.fori_loop(0, BPW // L, ext_body, jnp.int32(0))

        v0 = idxf[pl.ds(ebase, L)]
        p0 = plsc.load_gather(pos, [v0])
        is0 = lane == 0
        padi = jnp.sum(jnp.where(is0, v0, 0))
        padp = jnp.sum(jnp.where(is0, p0, 0))
        padiv = jnp.full((L,), padi, jnp.int32)
        padpv = jnp.full((L,), padp, jnp.int32)

        def pad_body(t, _):
            cidx1[pl.ds(n_win + t * L, L)] = padiv
            cpos1[pl.ds(n_win + t * L, L)] = padpv
            return 0

        lax.fori_loop(0, (BPW - n_win + L - 1) // L, pad_body, 0)

        for j in range(NCHUNK):
            for t in range(RCH // L):
                cidx2[j, pl.ds(t * L, L)] = cidx1[pl.ds(j * RCH + t * L, L)]
                cpos2[j, pl.ds(t * L, L)] = cpos1[pl.ds(j * RCH + t * L, L)]
        pltpu.sync_copy(cidx2, cidx_hbm.at[wid])
        pltpu.sync_copy(cpos2, cpos_hbm.at[wid])

    return winners_kernel


@functools.lru_cache(maxsize=None)
def _make_row_scatter(M, D, B):
    BPW = B // NW
    NCHUNK = BPW // RCH

    mesh = plsc.VectorSubcoreMesh(
        core_axis_name="c", subcore_axis_name="s", num_cores=NC, num_subcores=NS
    )

    @functools.partial(
        pl.kernel,
        out_type=(),
        mesh=mesh,
        compiler_params=pltpu.CompilerParams(needs_layout_passes=False),
        scratch_types=[
            pltpu.VMEM((NCHUNK, RCH), jnp.int32),
            pltpu.VMEM((NCHUNK, RCH), jnp.int32),
        ]
        + [pltpu.VMEM((RCH, D), jnp.float32) for _ in range(NCHUNK)]
        + [
            pltpu.SemaphoreType.DMA,
            pltpu.SemaphoreType.DMA,
        ],
    )
    def row_scatter_kernel(cidx_hbm, cpos_hbm, val_hbm, out_hbm, cidxv, cposv,
                           *rest):
        rows = rest[:NCHUNK]
        gsem, ssem = rest[NCHUNK:]
        c = lax.axis_index("c")
        s = lax.axis_index("s")
        wid = s * NC + c

        pltpu.sync_copy(cidx_hbm.at[wid], cidxv)
        pltpu.sync_copy(cpos_hbm.at[wid], cposv)

        gathers = [
            pltpu.async_copy(val_hbm.at[cposv.at[j]], rows[j], gsem)
            for j in range(NCHUNK)
        ]
        scatters = []
        for j in range(NCHUNK):
            gathers[j].wait()
            scatters.append(
                pltpu.async_copy(rows[j], out_hbm.at[cidxv.at[j]], ssem)
            )
        for sc_ in scatters:
            sc_.wait()

    return row_scatter_kernel


def kernel(mem, idx, val):
    M, D = mem.shape
    B = idx.shape[0]
    out_ref = jax.new_ref(mem)
    cidx, cpos = _make_winners(M, B)(idx)
    _make_row_scatter(M, D, B)(cidx, cpos, val, out_ref)
    return out_ref[...]
